# baseline (speedup 1.0000x reference)
import jax
import jax.numpy as jnp
from jax.experimental import pallas as pl
from jax.experimental.pallas import tpu as pltpu

_TOPN = 50
_QB = 64
_CB = 2048
_BIGF = 3.0e38
_BIGI = 2**30


def _make_body(n_chunks, mw):
    groups_per_chunk = _CB // 128

    def body(q_ref, ct_ref, w_ref, b_ref, od_ref, oi_ref, d_ref, m_ref):
        q = q_ref[...]
        w = w_ref[...]
        qp = jax.lax.dot_general(
            q, w, (((1,), (1,)), ((), ())), preferred_element_type=jnp.float32
        ) + b_ref[...]
        qsq = jnp.sum(qp * qp, axis=1, keepdims=True)

        if mw > n_chunks * groups_per_chunk:
            pad_w = mw - n_chunks * groups_per_chunk
            m_ref[:, n_chunks * groups_per_chunk :] = jnp.full(
                (_QB, pad_w), _BIGF, jnp.float32
            )

        for j in range(n_chunks):
            ctj = ct_ref[:, j * _CB : (j + 1) * _CB]
            csqj = jnp.sum(ctj * ctj, axis=0, keepdims=True)
            dotj = jnp.dot(qp, ctj, preferred_element_type=jnp.float32)
            dj = jnp.sqrt(jnp.maximum((qsq + csqj) - 2.0 * dotj, 0.0))
            dj3 = dj.reshape(_QB, groups_per_chunk, 128)
            d_ref[:, j * groups_per_chunk : (j + 1) * groups_per_chunk, :] = dj3
            m_ref[:, j * groups_per_chunk : (j + 1) * groups_per_chunk] = jnp.min(
                dj3, axis=2
            )

        lane_64 = jax.lax.broadcasted_iota(jnp.int32, (8, 64), 1)
        lane_64_1 = jax.lax.broadcasted_iota(jnp.int32, (1, 64), 1)
        lane_mw1 = jax.lax.broadcasted_iota(jnp.int32, (1, mw), 1)
        lane_128 = jax.lax.broadcasted_iota(jnp.int32, (1, 128), 1)

        for gg in range(8):
            base = 8 * gg

            def round_body(r, carry, base=base):
                mb = m_ref[base : base + 8, :]
                mm = jnp.min(mb, axis=1, keepdims=True)
                ob = od_ref[base : base + 8, :]
                od_ref[base : base + 8, :] = jnp.where(lane_64 == r, mm, ob)
                for qq in range(8):
                    qa = base + qq
                    mrow = jax.lax.slice(mb, (qq, 0), (qq + 1, mw))
                    mm_q = jnp.min(mrow)
                    g_q = jnp.min(
                        jnp.where(mrow == mm_q, lane_mw1, _BIGI)
                    )
                    row = d_ref[qa, pl.ds(g_q, 1), :]
                    l_q = jnp.min(jnp.where(row == mm_q, lane_128, _BIGI))
                    idx_q = g_q * 128 + l_q
                    irow = oi_ref[qa : qa + 1, :]
                    oi_ref[qa : qa + 1, :] = jnp.where(lane_64_1 == r, idx_q, irow)
                    masked = jnp.where(lane_128 == l_q, _BIGF, row)
                    d_ref[qa, pl.ds(g_q, 1), :] = masked
                    nm_q = jnp.min(masked)
                    m_ref[qa : qa + 1, :] = jnp.where(lane_mw1 == g_q, nm_q, mrow)
                return carry

            jax.lax.fori_loop(0, _TOPN, round_body, 0)

    return body


def kernel(query_embeddings, context_embeddings, W, b):
    nq, d = query_embeddings.shape
    k = context_embeddings.shape[0]
    kp = ((k + _CB - 1) // _CB) * _CB
    n_chunks = kp // _CB
    n_groups = kp // 128
    mw = ((n_groups + 127) // 128) * 128

    pad = jnp.full((kp - k, d), 1e15, jnp.float32)
    ct = jnp.concatenate([context_embeddings, pad], axis=0).T

    out_d, out_i = pl.pallas_call(
        _make_body(n_chunks, mw),
        grid=(nq // _QB,),
        in_specs=[
            pl.BlockSpec((_QB, d), lambda i: (i, 0)),
            pl.BlockSpec((d, kp), lambda i: (0, 0)),
            pl.BlockSpec((d, d), lambda i: (0, 0)),
            pl.BlockSpec((1, d), lambda i: (0, 0)),
        ],
        out_specs=[
            pl.BlockSpec((_QB, 64), lambda i: (i, 0)),
            pl.BlockSpec((_QB, 64), lambda i: (i, 0)),
        ],
        out_shape=[
            jax.ShapeDtypeStruct((nq, 64), jnp.float32),
            jax.ShapeDtypeStruct((nq, 64), jnp.int32),
        ],
        scratch_shapes=[
            pltpu.VMEM((_QB, n_groups, 128), jnp.float32),
            pltpu.VMEM((_QB, mw), jnp.float32),
        ],
    )(query_embeddings, ct, W, b.reshape(1, d))
    return (out_d[:, :_TOPN], out_i[:, :_TOPN])


# speedup vs baseline: 39.8773x; 39.8773x over previous
import jax
import jax.numpy as jnp
from jax.experimental import pallas as pl
from jax.experimental.pallas import tpu as pltpu

_TOPN = 50
_QB = 64
_CB = 2048
_BIGF = 3.0e38
_BIGI = 2**30


def _make_body(n_chunks, mw):
    groups_per_chunk = _CB // 128

    def body(q_ref, ct_ref, w_ref, b_ref, od_ref, oi_ref, d_ref, m_ref, r_ref):
        q = q_ref[...]
        w = w_ref[...]
        qp = jax.lax.dot_general(
            q, w, (((1,), (1,)), ((), ())), preferred_element_type=jnp.float32
        ) + b_ref[...]
        qsq = jnp.sum(qp * qp, axis=1, keepdims=True)

        if mw > n_chunks * groups_per_chunk:
            pad_w = mw - n_chunks * groups_per_chunk
            m_ref[:, n_chunks * groups_per_chunk :] = jnp.full(
                (_QB, pad_w), _BIGF, jnp.float32
            )

        for j in range(n_chunks):
            ctj = ct_ref[:, j * _CB : (j + 1) * _CB]
            csqj = jnp.sum(ctj * ctj, axis=0, keepdims=True)
            dotj = jnp.dot(qp, ctj, preferred_element_type=jnp.float32)
            dj = jnp.maximum((qsq + csqj) - 2.0 * dotj, 0.0)
            dj3 = dj.reshape(_QB, groups_per_chunk, 128)
            d_ref[:, j * groups_per_chunk : (j + 1) * groups_per_chunk, :] = dj3
            m_ref[:, j * groups_per_chunk : (j + 1) * groups_per_chunk] = jnp.min(
                dj3, axis=2
            )

        lane_out = jax.lax.broadcasted_iota(jnp.int32, (_QB, 64), 1)
        lane_mw = jax.lax.broadcasted_iota(jnp.int32, (_QB, mw), 1)
        lane_128 = jax.lax.broadcasted_iota(jnp.int32, (_QB, 128), 1)

        def round_body(r, carry):
            od_acc, oi_acc = carry
            mb = m_ref[...]
            mm = jnp.min(mb, axis=1, keepdims=True)
            g = jnp.min(
                jnp.where(mb == mm, lane_mw, _BIGI), axis=1, keepdims=True
            )
            gq = []
            for qq in range(_QB):
                g_q = jnp.min(jax.lax.slice(g, (qq, 0), (qq + 1, 1)))
                gq.append(g_q)
                r_ref[qq : qq + 1, :] = d_ref[qq, pl.ds(g_q, 1), :]
            rows = r_ref[...]
            l = jnp.min(
                jnp.where(rows == mm, lane_128, _BIGI), axis=1, keepdims=True
            )
            idx = g * 128 + l
            od_acc = jnp.where(lane_out == r, mm, od_acc)
            oi_acc = jnp.where(lane_out == r, idx, oi_acc)
            masked = jnp.where(lane_128 == l, _BIGF, rows)
            r_ref[...] = masked
            for qq in range(_QB):
                d_ref[qq, pl.ds(gq[qq], 1), :] = r_ref[qq : qq + 1, :]
            nm = jnp.min(masked, axis=1, keepdims=True)
            m_ref[...] = jnp.where(lane_mw == g, nm, mb)
            return od_acc, oi_acc

        od0 = jnp.zeros((_QB, 64), jnp.float32)
        oi0 = jnp.zeros((_QB, 64), jnp.int32)
        od_acc, oi_acc = jax.lax.fori_loop(0, _TOPN, round_body, (od0, oi0))
        od_ref[...] = jnp.sqrt(od_acc)
        oi_ref[...] = oi_acc

    return body


def kernel(query_embeddings, context_embeddings, W, b):
    nq, d = query_embeddings.shape
    k = context_embeddings.shape[0]
    kp = ((k + _CB - 1) // _CB) * _CB
    n_chunks = kp // _CB
    n_groups = kp // 128
    mw = ((n_groups + 127) // 128) * 128

    pad = jnp.full((kp - k, d), 1e15, jnp.float32)
    ct = jnp.concatenate([context_embeddings, pad], axis=0).T

    out_d, out_i = pl.pallas_call(
        _make_body(n_chunks, mw),
        grid=(nq // _QB,),
        in_specs=[
            pl.BlockSpec((_QB, d), lambda i: (i, 0)),
            pl.BlockSpec((d, kp), lambda i: (0, 0)),
            pl.BlockSpec((d, d), lambda i: (0, 0)),
            pl.BlockSpec((1, d), lambda i: (0, 0)),
        ],
        out_specs=[
            pl.BlockSpec((_QB, 64), lambda i: (i, 0)),
            pl.BlockSpec((_QB, 64), lambda i: (i, 0)),
        ],
        out_shape=[
            jax.ShapeDtypeStruct((nq, 64), jnp.float32),
            jax.ShapeDtypeStruct((nq, 64), jnp.int32),
        ],
        scratch_shapes=[
            pltpu.VMEM((_QB, n_groups, 128), jnp.float32),
            pltpu.VMEM((_QB, mw), jnp.float32),
            pltpu.VMEM((_QB, 128), jnp.float32),
        ],
    )(query_embeddings, ct, W, b.reshape(1, d))
    return (out_d[:, :_TOPN], out_i[:, :_TOPN])


# speedup vs baseline: 41.1624x; 1.0322x over previous
import jax
import jax.numpy as jnp
from jax.experimental import pallas as pl
from jax.experimental.pallas import tpu as pltpu

_TOPN = 50
_QB = 64
_CB = 2048
_BIGF = 3.0e38
_BIGI = 2**30


def _make_body(n_chunks, mw):
    groups_per_chunk = _CB // 128

    def body(q_ref, ct_ref, w_ref, b_ref, od_ref, oi_ref, d_ref, m_ref, r_ref):
        q = q_ref[...]
        w = w_ref[...]
        qp = jax.lax.dot_general(
            q, w, (((1,), (1,)), ((), ())), preferred_element_type=jnp.float32
        ) + b_ref[...]
        qsq = jnp.sum(qp * qp, axis=1, keepdims=True)

        if mw > n_chunks * groups_per_chunk:
            pad_w = mw - n_chunks * groups_per_chunk
            m_ref[:, n_chunks * groups_per_chunk :] = jnp.full(
                (_QB, pad_w), _BIGF, jnp.float32
            )

        for j in range(n_chunks):
            ctj = ct_ref[:, j * _CB : (j + 1) * _CB]
            csqj = jnp.sum(ctj * ctj, axis=0, keepdims=True)
            dotj = jnp.dot(qp, ctj, preferred_element_type=jnp.float32)
            dj = jnp.maximum((qsq + csqj) - 2.0 * dotj, 0.0)
            mins = []
            for g in range(groups_per_chunk):
                sl = jax.lax.slice(dj, (0, g * 128), (_QB, (g + 1) * 128))
                d_ref[j * groups_per_chunk + g, :, :] = sl
                mins.append(jnp.min(sl, axis=1, keepdims=True))
            m_ref[:, j * groups_per_chunk : (j + 1) * groups_per_chunk] = (
                jnp.concatenate(mins, axis=1)
            )

        lane_out = jax.lax.broadcasted_iota(jnp.int32, (_QB, 64), 1)
        lane_mw = jax.lax.broadcasted_iota(jnp.int32, (_QB, mw), 1)
        lane_128 = jax.lax.broadcasted_iota(jnp.int32, (_QB, 128), 1)

        def round_body(r, carry):
            od_acc, oi_acc = carry
            mb = m_ref[...]
            mm = jnp.min(mb, axis=1, keepdims=True)
            g = jnp.min(
                jnp.where(mb == mm, lane_mw, _BIGI), axis=1, keepdims=True
            )
            gq = []
            for qq in range(_QB):
                g_q = jnp.min(jax.lax.slice(g, (qq, 0), (qq + 1, 1)))
                gq.append(g_q)
                r_ref[qq : qq + 1, :] = d_ref[pl.ds(g_q, 1), qq, :]
            rows = r_ref[...]
            l = jnp.min(
                jnp.where(rows == mm, lane_128, _BIGI), axis=1, keepdims=True
            )
            idx = g * 128 + l
            od_acc = jnp.where(lane_out == r, mm, od_acc)
            oi_acc = jnp.where(lane_out == r, idx, oi_acc)
            masked = jnp.where(lane_128 == l, _BIGF, rows)
            r_ref[...] = masked
            for qq in range(_QB):
                d_ref[pl.ds(gq[qq], 1), qq, :] = r_ref[qq : qq + 1, :]
            nm = jnp.min(masked, axis=1, keepdims=True)
            m_ref[...] = jnp.where(lane_mw == g, nm, mb)
            return od_acc, oi_acc

        od0 = jnp.zeros((_QB, 64), jnp.float32)
        oi0 = jnp.zeros((_QB, 64), jnp.int32)
        od_acc, oi_acc = jax.lax.fori_loop(0, _TOPN, round_body, (od0, oi0))
        od_ref[...] = jnp.sqrt(od_acc)
        oi_ref[...] = oi_acc

    return body


def kernel(query_embeddings, context_embeddings, W, b):
    nq, d = query_embeddings.shape
    k = context_embeddings.shape[0]
    kp = ((k + _CB - 1) // _CB) * _CB
    n_chunks = kp // _CB
    n_groups = kp // 128
    mw = ((n_groups + 127) // 128) * 128

    pad = jnp.full((kp - k, d), 1e15, jnp.float32)
    ct = jnp.concatenate([context_embeddings, pad], axis=0).T

    out_d, out_i = pl.pallas_call(
        _make_body(n_chunks, mw),
        grid=(nq // _QB,),
        in_specs=[
            pl.BlockSpec((_QB, d), lambda i: (i, 0)),
            pl.BlockSpec((d, kp), lambda i: (0, 0)),
            pl.BlockSpec((d, d), lambda i: (0, 0)),
            pl.BlockSpec((1, d), lambda i: (0, 0)),
        ],
        out_specs=[
            pl.BlockSpec((_QB, 64), lambda i: (i, 0)),
            pl.BlockSpec((_QB, 64), lambda i: (i, 0)),
        ],
        out_shape=[
            jax.ShapeDtypeStruct((nq, 64), jnp.float32),
            jax.ShapeDtypeStruct((nq, 64), jnp.int32),
        ],
        scratch_shapes=[
            pltpu.VMEM((n_groups, _QB, 128), jnp.float32),
            pltpu.VMEM((_QB, mw), jnp.float32),
            pltpu.VMEM((_QB, 128), jnp.float32),
        ],
    )(query_embeddings, ct, W, b.reshape(1, d))
    return (out_d[:, :_TOPN], out_i[:, :_TOPN])


# speedup vs baseline: 49.1128x; 1.1931x over previous
import jax
import jax.numpy as jnp
from jax.experimental import pallas as pl
from jax.experimental.pallas import tpu as pltpu

_TOPN = 50
_QB = 64
_CB = 2048
_BIGF = 3.0e38
_BIGI = 2**30


def _make_body(n_chunks, mw):
    groups_per_chunk = _CB // 128

    def body(
        q_ref, ct_ref, w_ref, b_ref, od_ref, oi_ref, d_ref, m_ref, r_ref, r2_ref
    ):
        q = q_ref[...]
        w = w_ref[...]
        qp = jax.lax.dot_general(
            q, w, (((1,), (1,)), ((), ())), preferred_element_type=jnp.float32
        ) + b_ref[...]
        qsq = jnp.sum(qp * qp, axis=1, keepdims=True)

        if mw > n_chunks * groups_per_chunk:
            pad_w = mw - n_chunks * groups_per_chunk
            m_ref[:, n_chunks * groups_per_chunk :] = jnp.full(
                (_QB, pad_w), _BIGF, jnp.float32
            )

        for j in range(n_chunks):
            ctj = ct_ref[:, j * _CB : (j + 1) * _CB]
            csqj = jnp.sum(ctj * ctj, axis=0, keepdims=True)
            dotj = jnp.dot(qp, ctj, preferred_element_type=jnp.float32)
            dj = jnp.maximum((qsq + csqj) - 2.0 * dotj, 0.0)
            mins = []
            for g in range(groups_per_chunk):
                sl = jax.lax.slice(dj, (0, g * 128), (_QB, (g + 1) * 128))
                d_ref[j * groups_per_chunk + g, :, :] = sl
                mins.append(jnp.min(sl, axis=1, keepdims=True))
            m_ref[:, j * groups_per_chunk : (j + 1) * groups_per_chunk] = (
                jnp.concatenate(mins, axis=1)
            )

        lane_out = jax.lax.broadcasted_iota(jnp.int32, (_QB, 64), 1)
        lane_mw = jax.lax.broadcasted_iota(jnp.int32, (_QB, mw), 1)
        lane_128 = jax.lax.broadcasted_iota(jnp.int32, (_QB, 128), 1)

        def round_body(r, carry):
            od_acc, oi_acc, mb = carry
            mm1 = jnp.min(mb, axis=1, keepdims=True)
            g1 = jnp.min(
                jnp.where(mb == mm1, lane_mw, _BIGI), axis=1, keepdims=True
            )
            mb_ex = jnp.where(lane_mw == g1, _BIGF, mb)
            mm2 = jnp.min(mb_ex, axis=1, keepdims=True)
            g2 = jnp.min(
                jnp.where(mb_ex == mm2, lane_mw, _BIGI), axis=1, keepdims=True
            )
            g1s, g2s = [], []
            for qq in range(_QB):
                g1_q = jnp.min(jax.lax.slice(g1, (qq, 0), (qq + 1, 1)))
                g1s.append(g1_q)
                r_ref[qq : qq + 1, :] = d_ref[pl.ds(g1_q, 1), qq, :]
            for qq in range(_QB):
                g2_q = jnp.min(jax.lax.slice(g2, (qq, 0), (qq + 1, 1)))
                g2s.append(g2_q)
                r2_ref[qq : qq + 1, :] = d_ref[pl.ds(g2_q, 1), qq, :]
            rows1 = r_ref[...]
            rows2 = r2_ref[...]
            l1 = jnp.min(
                jnp.where(rows1 == mm1, lane_128, _BIGI), axis=1, keepdims=True
            )
            idx1 = g1 * 128 + l1
            masked1 = jnp.where(lane_128 == l1, _BIGF, rows1)
            nm1 = jnp.min(masked1, axis=1, keepdims=True)
            l1b = jnp.min(
                jnp.where(masked1 == nm1, lane_128, _BIGI), axis=1, keepdims=True
            )
            idx_a = g1 * 128 + l1b
            l2 = jnp.min(
                jnp.where(rows2 == mm2, lane_128, _BIGI), axis=1, keepdims=True
            )
            idx_b = g2 * 128 + l2
            from_a = (nm1 < mm2) | ((nm1 == mm2) & (idx_a < idx_b))
            e2 = jnp.where(from_a, nm1, mm2)
            i2 = jnp.where(from_a, idx_a, idx_b)
            od_acc = jnp.where(
                lane_out == 2 * r, mm1, jnp.where(lane_out == 2 * r + 1, e2, od_acc)
            )
            oi_acc = jnp.where(
                lane_out == 2 * r, idx1, jnp.where(lane_out == 2 * r + 1, i2, oi_acc)
            )
            masked1f = jnp.where(from_a & (lane_128 == l1b), _BIGF, masked1)
            masked2f = jnp.where((~from_a) & (lane_128 == l2), _BIGF, rows2)
            r_ref[...] = masked1f
            r2_ref[...] = masked2f
            for qq in range(_QB):
                d_ref[pl.ds(g1s[qq], 1), qq, :] = r_ref[qq : qq + 1, :]
            for qq in range(_QB):
                d_ref[pl.ds(g2s[qq], 1), qq, :] = r2_ref[qq : qq + 1, :]
            nm1f = jnp.min(masked1f, axis=1, keepdims=True)
            nm2f = jnp.min(masked2f, axis=1, keepdims=True)
            mb = jnp.where(
                lane_mw == g1, nm1f, jnp.where(lane_mw == g2, nm2f, mb)
            )
            return od_acc, oi_acc, mb

        od0 = jnp.zeros((_QB, 64), jnp.float32)
        oi0 = jnp.zeros((_QB, 64), jnp.int32)
        od_acc, oi_acc, _ = jax.lax.fori_loop(
            0, _TOPN // 2, round_body, (od0, oi0, m_ref[...])
        )
        od_ref[...] = jnp.sqrt(od_acc)
        oi_ref[...] = oi_acc

    return body


def kernel(query_embeddings, context_embeddings, W, b):
    nq, d = query_embeddings.shape
    k = context_embeddings.shape[0]
    kp = ((k + _CB - 1) // _CB) * _CB
    n_chunks = kp // _CB
    n_groups = kp // 128
    mw = ((n_groups + 127) // 128) * 128

    pad = jnp.full((kp - k, d), 1e15, jnp.float32)
    ct = jnp.concatenate([context_embeddings, pad], axis=0).T

    out_d, out_i = pl.pallas_call(
        _make_body(n_chunks, mw),
        grid=(nq // _QB,),
        in_specs=[
            pl.BlockSpec((_QB, d), lambda i: (i, 0)),
            pl.BlockSpec((d, kp), lambda i: (0, 0)),
            pl.BlockSpec((d, d), lambda i: (0, 0)),
            pl.BlockSpec((1, d), lambda i: (0, 0)),
        ],
        out_specs=[
            pl.BlockSpec((_QB, 64), lambda i: (i, 0)),
            pl.BlockSpec((_QB, 64), lambda i: (i, 0)),
        ],
        out_shape=[
            jax.ShapeDtypeStruct((nq, 64), jnp.float32),
            jax.ShapeDtypeStruct((nq, 64), jnp.int32),
        ],
        scratch_shapes=[
            pltpu.VMEM((n_groups, _QB, 128), jnp.float32),
            pltpu.VMEM((_QB, mw), jnp.float32),
            pltpu.VMEM((_QB, 128), jnp.float32),
            pltpu.VMEM((_QB, 128), jnp.float32),
        ],
    )(query_embeddings, ct, W, b.reshape(1, d))
    return (out_d[:, :_TOPN], out_i[:, :_TOPN])
